# baseline (speedup 1.0000x reference)
import functools
import jax
import jax.numpy as jnp
from jax import lax
from jax.experimental import pallas as pl
from jax.experimental.pallas import tpu as pltpu
from jax.experimental.pallas import tpu_sc as plsc

D_MODEL = 64
ROWS = 4096
COLS = 200
B = ROWS * COLS
NC = 2
NS = 16
NW = NC * NS
BPW = B // NW
CH = 128
NCH = BPW // CH
SCALE = 8.0

_mesh = plsc.VectorSubcoreMesh(core_axis_name="c", subcore_axis_name="s")


@functools.partial(
    pl.kernel,
    mesh=_mesh,
    out_type=jax.ShapeDtypeStruct((B, D_MODEL), jnp.float32),
    scratch_types=[
        pltpu.VMEM((CH,), jnp.int32),
        pltpu.VMEM((CH, D_MODEL), jnp.float32),
        pltpu.SemaphoreType.DMA,
    ],
    compiler_params=pltpu.CompilerParams(use_tc_tiling_on_sc=False),
)
def _embed(x_hbm, table_hbm, out_hbm, idx_v, rows_v, sem):
    wid = lax.axis_index("s") * NC + lax.axis_index("c")
    base = wid * BPW

    def chunk_body(c, carry):
        off = pl.multiple_of(base + c * CH, CH)
        pltpu.sync_copy(x_hbm.at[pl.ds(off, CH)], idx_v)
        pltpu.async_copy(table_hbm.at[idx_v], rows_v, sem).wait()

        def row_body(i, carry2):
            for j in range(D_MODEL // 16):
                sl = pl.ds(j * 16, 16)
                rows_v[i, sl] = rows_v[i, sl] * SCALE
            return carry2

        lax.fori_loop(0, CH, row_body, 0)
        pltpu.sync_copy(rows_v, out_hbm.at[pl.ds(off, CH)])
        return carry

    lax.fori_loop(0, NCH, chunk_body, 0)


def kernel(x, table):
    flat = x.reshape(-1).astype(jnp.int32)
    out = _embed(flat, table)
    return out.reshape(ROWS, COLS, D_MODEL)


# speedup vs baseline: 1.2777x; 1.2777x over previous
import functools
import jax
import jax.numpy as jnp
from jax import lax
from jax.experimental import pallas as pl
from jax.experimental.pallas import tpu as pltpu
from jax.experimental.pallas import tpu_sc as plsc

D_MODEL = 64
ROWS = 4096
COLS = 200
B = ROWS * COLS
NC = 2
NS = 16
NW = NC * NS
BPW = B // NW
CH = 128
NCH = BPW // CH
NBUF = 4
SCALE = 8.0

_mesh = plsc.VectorSubcoreMesh(core_axis_name="c", subcore_axis_name="s")


@functools.partial(
    pl.kernel,
    mesh=_mesh,
    out_type=jax.ShapeDtypeStruct((B, D_MODEL), jnp.float32),
    scratch_types=[
        pltpu.VMEM((NCH, CH), jnp.int32),
        [pltpu.VMEM((CH, D_MODEL), jnp.float32) for _ in range(NBUF)],
        [pltpu.VMEM((CH, D_MODEL), jnp.float32) for _ in range(NBUF)],
        [pltpu.SemaphoreType.DMA for _ in range(NBUF)],
        [pltpu.SemaphoreType.DMA for _ in range(NBUF)],
    ],
    compiler_params=pltpu.CompilerParams(use_tc_tiling_on_sc=False),
)
def _embed(x_hbm, table_hbm, out_hbm, idx_v, gbufs, sbufs, gsems, ssems):
    wid = lax.axis_index("s") * NC + lax.axis_index("c")
    base = wid * BPW

    pltpu.sync_copy(x_hbm.at[wid], idx_v)

    def fire_gather(g, b):
        pltpu.async_copy(table_hbm.at[idx_v.at[g]], gbufs[b], gsems[b])

    def wait_gather(b):
        pltpu.make_async_copy(table_hbm.at[idx_v.at[0]], gbufs[b], gsems[b]).wait()

    def fire_store(g, b):
        dst = out_hbm.at[pl.ds(base + g * CH, CH)]
        pltpu.async_copy(sbufs[b], dst, ssems[b])

    def wait_store(b):
        dst = out_hbm.at[pl.ds(base, CH)]
        pltpu.make_async_copy(sbufs[b], dst, ssems[b]).wait()

    def scale_chunk(b):
        @plsc.parallel_loop(0, CH, 1, unroll=4)
        def _(i):
            for j in range(D_MODEL // 16):
                sl = pl.ds(j * 16, 16)
                sbufs[b][i, sl] = gbufs[b][i, sl] * SCALE

    for b in range(NBUF):
        fire_gather(b, b)

    for b in range(NBUF):
        wait_gather(b)
        scale_chunk(b)
        fire_store(b, b)
        fire_gather(b + NBUF, b)

    def outer(o, carry):
        g0 = o * NBUF
        for b in range(NBUF):
            g = g0 + b
            wait_gather(b)
            wait_store(b)
            scale_chunk(b)
            fire_store(g, b)

            @pl.when(g + NBUF < NCH)
            def _():
                fire_gather(g + NBUF, b)

        return carry

    lax.fori_loop(1, NCH // NBUF, outer, 0)

    for b in range(NBUF):
        wait_store(b)


def kernel(x, table):
    x3 = x.reshape(NW, NCH, CH).astype(jnp.int32)
    out = _embed(x3, table)
    return out.reshape(ROWS, COLS, D_MODEL)
